# baseline (speedup 1.0000x reference)
import functools

import jax
import jax.numpy as jnp
from jax import lax
from jax.experimental import pallas as pl
from jax.experimental.pallas import tpu as pltpu
from jax.experimental.pallas import tpu_sc as plsc

LANES = 16
XCHUNKS = (256, 256)
N_XCHUNKS = len(XCHUNKS)


def _tw_tc_kernel(tableT_ref, wT_ref, b_ref, out_ref, *, inv_l):
    tT = tableT_ref[...]
    wT = wT_ref[...]
    tw = jnp.dot(wT, tT, preferred_element_type=jnp.float32)
    out_ref[...] = (tw + b_ref[0]) * inv_l


def _make_sc_lookup(V, B, L, n_workers):
    sents_per_worker = B // n_workers
    mesh = plsc.VectorSubcoreMesh(core_axis_name="c", subcore_axis_name="s")

    @functools.partial(
        pl.kernel,
        mesh=mesh,
        out_type=jax.ShapeDtypeStruct((B,), jnp.float32),
        scratch_types=[
            pltpu.VMEM((L, sents_per_worker), jnp.int32),
            pltpu.VMEM((V,), jnp.float32),
            pltpu.VMEM((sents_per_worker,), jnp.float32),
            [pltpu.SemaphoreType.DMA] * N_XCHUNKS,
        ],
        compiler_params=pltpu.CompilerParams(needs_layout_passes=False),
    )
    def sc_lookup(xt_hbm, tw_hbm, out_hbm, idx_v, tw_v, out_v, sems):
        n_cores = 2
        wid = lax.axis_index("s") * n_cores + lax.axis_index("c")
        base_s = wid * sents_per_worker

        offs = [sum(XCHUNKS[:c]) for c in range(N_XCHUNKS)]
        copies = [
            pltpu.async_copy(
                xt_hbm.at[:, pl.ds(base_s + offs[c], XCHUNKS[c])],
                idx_v.at[:, pl.ds(offs[c], XCHUNKS[c])],
                sems[c])
            for c in range(N_XCHUNKS)
        ]
        pltpu.sync_copy(tw_hbm.at[0], tw_v)

        def blk_body(blk, _):
            s0 = blk * (2 * LANES)

            def body(t, accs):
                a0, a1 = accs
                xv0 = idx_v[t, pl.ds(s0, LANES)]
                xv1 = idx_v[t, pl.ds(s0 + LANES, LANES)]
                tv0 = plsc.load_gather(tw_v, [xv0])
                tv1 = plsc.load_gather(tw_v, [xv1])
                return a0 + tv0, a1 + tv1

            zero = jnp.zeros((LANES,), jnp.float32)
            a0, a1 = lax.fori_loop(0, L, body, (zero, zero),
                                   unroll=8)
            out_v[pl.ds(s0, LANES)] = 1.0 / (1.0 + jnp.exp(-a0))
            out_v[pl.ds(s0 + LANES, LANES)] = 1.0 / (1.0 + jnp.exp(-a1))
            return 0

        for c in range(N_XCHUNKS):
            copies[c].wait()
            lax.fori_loop(offs[c] // (2 * LANES),
                          (offs[c] + XCHUNKS[c]) // (2 * LANES),
                          blk_body, 0)

        pltpu.sync_copy(out_v, out_hbm.at[pl.ds(base_s, sents_per_worker)])

    return sc_lookup


def kernel(x, table, W, b):
    B, L = x.shape
    V, EMB = table.shape

    tw = pl.pallas_call(
        functools.partial(_tw_tc_kernel, inv_l=1.0 / L),
        out_shape=jax.ShapeDtypeStruct((1, V), jnp.float32),
    )(table.T, W.T, b)

    out = _make_sc_lookup(V, B, L, 32)(x.T.astype(jnp.int32), tw)
    return out.reshape(B, 1)


# speedup vs baseline: 1.0363x; 1.0363x over previous
import functools

import jax
import jax.numpy as jnp
from jax import lax
from jax.experimental import pallas as pl
from jax.experimental.pallas import tpu as pltpu
from jax.experimental.pallas import tpu_sc as plsc

LANES = 16
XCHUNKS = (128, 384)
N_XCHUNKS = len(XCHUNKS)


def _tw_tc_kernel(tableT_ref, wT_ref, b_ref, out_ref, *, inv_l):
    tT = tableT_ref[...]
    wT = wT_ref[...]
    tw = jnp.dot(wT, tT, preferred_element_type=jnp.float32)
    out_ref[...] = (tw + b_ref[0]) * inv_l


def _make_sc_lookup(V, B, L, n_workers):
    sents_per_worker = B // n_workers
    mesh = plsc.VectorSubcoreMesh(core_axis_name="c", subcore_axis_name="s")

    @functools.partial(
        pl.kernel,
        mesh=mesh,
        out_type=jax.ShapeDtypeStruct((B,), jnp.float32),
        scratch_types=[
            pltpu.VMEM((L, sents_per_worker), jnp.int32),
            pltpu.VMEM((V,), jnp.float32),
            pltpu.VMEM((sents_per_worker,), jnp.float32),
            [pltpu.SemaphoreType.DMA] * N_XCHUNKS,
        ],
        compiler_params=pltpu.CompilerParams(needs_layout_passes=False),
    )
    def sc_lookup(xt_hbm, tw_hbm, out_hbm, idx_v, tw_v, out_v, sems):
        n_cores = 2
        wid = lax.axis_index("s") * n_cores + lax.axis_index("c")
        base_s = wid * sents_per_worker

        offs = [sum(XCHUNKS[:c]) for c in range(N_XCHUNKS)]
        copies = [
            pltpu.async_copy(
                xt_hbm.at[:, pl.ds(base_s + offs[c], XCHUNKS[c])],
                idx_v.at[:, pl.ds(offs[c], XCHUNKS[c])],
                sems[c])
            for c in range(N_XCHUNKS)
        ]
        pltpu.sync_copy(tw_hbm.at[0], tw_v)

        def blk_body(blk, _):
            s0 = blk * (2 * LANES)

            def body(t, accs):
                a0, a1 = accs
                xv0 = idx_v[t, pl.ds(s0, LANES)]
                xv1 = idx_v[t, pl.ds(s0 + LANES, LANES)]
                tv0 = plsc.load_gather(tw_v, [xv0])
                tv1 = plsc.load_gather(tw_v, [xv1])
                return a0 + tv0, a1 + tv1

            zero = jnp.zeros((LANES,), jnp.float32)
            a0, a1 = lax.fori_loop(0, L, body, (zero, zero),
                                   unroll=8)
            out_v[pl.ds(s0, LANES)] = 1.0 / (1.0 + jnp.exp(-a0))
            out_v[pl.ds(s0 + LANES, LANES)] = 1.0 / (1.0 + jnp.exp(-a1))
            return 0

        for c in range(N_XCHUNKS):
            copies[c].wait()
            lax.fori_loop(offs[c] // (2 * LANES),
                          (offs[c] + XCHUNKS[c]) // (2 * LANES),
                          blk_body, 0)

        pltpu.sync_copy(out_v, out_hbm.at[pl.ds(base_s, sents_per_worker)])

    return sc_lookup


def kernel(x, table, W, b):
    B, L = x.shape
    V, EMB = table.shape

    tw = pl.pallas_call(
        functools.partial(_tw_tc_kernel, inv_l=1.0 / L),
        out_shape=jax.ShapeDtypeStruct((1, V), jnp.float32),
    )(table.T, W.T, b)

    out = _make_sc_lookup(V, B, L, 32)(x.T.astype(jnp.int32), tw)
    return out.reshape(B, 1)
